# baseline (speedup 1.0000x reference)
import functools

import jax
import jax.numpy as jnp
from jax.experimental import pallas as pl

FREQ = 256
PHI = 0.7


def _mm_kernel(a_ref, b_ref, o_ref):
    o_ref[...] = jax.lax.dot_general(
        a_ref[...], b_ref[...], (((1,), (0,)), ((), ())),
        preferred_element_type=jnp.float32,
        precision=jax.lax.Precision.HIGHEST,
    )


@functools.partial(jax.jit, static_argnames=("bm", "bn"))
def _pmm(a, b, bm=512, bn=512):
    M, K = a.shape
    K2, N = b.shape
    assert K == K2
    return pl.pallas_call(
        _mm_kernel,
        grid=(M // bm, N // bn),
        in_specs=[
            pl.BlockSpec((bm, K), lambda i, j: (i, 0)),
            pl.BlockSpec((K, bn), lambda i, j: (0, j)),
        ],
        out_specs=pl.BlockSpec((bm, bn), lambda i, j: (i, j)),
        out_shape=jax.ShapeDtypeStruct((M, N), jnp.float32),
    )(a, b)


def kernel(R):
    user_num, item_num = R.shape
    du = R.sum(axis=1)
    di = R.sum(axis=0)
    du_inv = jnp.power(du, -0.5)
    di_inv = jnp.power(di, -0.5)
    di_pos = jnp.power(di, 0.5)
    Ru = du_inv[:, None] * R
    Ri = R * di_inv[None, :]
    Cu = _pmm(Ri, Ri.T)
    Ci = _pmm(Ru.T, Ru)
    R_post = Ru * di_inv[None, :]
    Ci0 = _pmm(R_post.T, R_post)
    Cu0 = _pmm(R_post, R_post.T)
    A = jnp.block([[Cu, R], [R.T, Ci]])
    I_mat = jnp.eye(user_num + item_num, dtype=R.dtype)
    dA = A.sum(axis=0)
    dA_inv = jnp.power(dA, -0.5)
    A_norm = dA_inv[:, None] * A * dA_inv[None, :]
    L_norm = I_mat - A_norm
    val, vec = jnp.linalg.eigh(L_norm)
    vec = vec[:, :FREQ]
    R_b = jnp.concatenate([Cu0, R], axis=1)
    d_rb = R_b.sum(axis=0)
    d_rb_inv = jnp.power(d_rb, -0.5)
    d_rb_pos = jnp.power(d_rb, 0.5)
    P0 = _pmm(R * di_inv[None, :], Ci0) * di_pos[None, :]
    P1 = _pmm(R_b * d_rb_inv[None, :], vec, bn=FREQ)
    P11 = vec.T * d_rb_pos[None, :]
    P1 = _pmm(P1, P11, bm=512)[:, user_num:]
    P = PHI * P0 + (1.0 - PHI) * P1
    return P


# speedup vs baseline: 13.4101x; 13.4101x over previous
import functools

import jax
import jax.numpy as jnp
from jax.experimental import pallas as pl

FREQ = 256
PHI = 0.7
M_SUB = 384
ROUNDS = 3
DEG = 20


def _mm_kernel(a_ref, b_ref, o_ref):
    o_ref[...] = jax.lax.dot_general(
        a_ref[...], b_ref[...], (((1,), (0,)), ((), ())),
        preferred_element_type=jnp.float32,
        precision=jax.lax.Precision.HIGHEST,
    )


def _mma_kernel(a_ref, b_ref, t_ref, o_ref):
    o_ref[...] = t_ref[...] + jax.lax.dot_general(
        a_ref[...], b_ref[...], (((1,), (0,)), ((), ())),
        preferred_element_type=jnp.float32,
        precision=jax.lax.Precision.HIGHEST,
    )


@functools.partial(jax.jit, static_argnames=("bm", "bn"))
def _pmm(a, b, bm=512, bn=512):
    M, K = a.shape
    _, N = b.shape
    bn = min(bn, N)
    return pl.pallas_call(
        _mm_kernel,
        grid=(M // bm, N // bn),
        in_specs=[
            pl.BlockSpec((bm, K), lambda i, j: (i, 0)),
            pl.BlockSpec((K, bn), lambda i, j: (0, j)),
        ],
        out_specs=pl.BlockSpec((bm, bn), lambda i, j: (i, j)),
        out_shape=jax.ShapeDtypeStruct((M, N), jnp.float32),
    )(a, b)


@functools.partial(jax.jit, static_argnames=("bm",))
def _pmma(a, b, t, bm=512):
    M, K = a.shape
    _, N = b.shape
    return pl.pallas_call(
        _mma_kernel,
        grid=(M // bm,),
        in_specs=[
            pl.BlockSpec((bm, K), lambda i: (i, 0)),
            pl.BlockSpec((K, N), lambda i: (0, 0)),
            pl.BlockSpec((bm, N), lambda i: (i, 0)),
        ],
        out_specs=pl.BlockSpec((bm, N), lambda i: (i, 0)),
        out_shape=jax.ShapeDtypeStruct((M, N), jnp.float32),
    )(a, b, t)


def _apply_deflated(An, v1, X):
    w = v1 @ X
    T = -v1[:, None] * w[None, :]
    return _pmma(An, X, T)


def _cholqr2(X):
    for _ in range(2):
        G = _pmm(X.T, X, bm=M_SUB, bn=M_SUB)
        L = jnp.linalg.cholesky(G)
        Linv = jax.lax.linalg.triangular_solve(
            L, jnp.eye(M_SUB, dtype=jnp.float32), lower=True, left_side=True)
        X = _pmm(X, Linv.T, bm=512, bn=M_SUB)
    return X


def kernel(R):
    user_num, item_num = R.shape
    n_tot = user_num + item_num
    du = R.sum(axis=1)
    di = R.sum(axis=0)
    du_inv = du ** -0.5
    di_inv = di ** -0.5
    di_pos = di ** 0.5
    Ru = du_inv[:, None] * R
    Ri = R * di_inv[None, :]
    R_post = Ru * di_inv[None, :]

    Cu = _pmm(Ri, Ri.T)
    Ci = _pmm(Ru.T, Ru)
    Ci0 = _pmm(R_post.T, R_post)
    Cu0 = _pmm(R_post, R_post.T)
    A = jnp.block([[Cu, R], [R.T, Ci]])
    dA = A.sum(axis=0)
    dA_inv = dA ** -0.5
    An = dA_inv[:, None] * A * dA_inv[None, :]
    v1 = jnp.sqrt(dA)
    v1 = v1 / jnp.linalg.norm(v1)

    X = jax.random.normal(jax.random.key(0), (n_tot, M_SUB), dtype=jnp.float32)
    X = _cholqr2(X)
    for r in range(ROUNDS):
        AX = _apply_deflated(An, v1, X)
        S = _pmm(X.T, AX, bm=M_SUB, bn=M_SUB)
        th = jnp.linalg.eigvalsh(0.5 * (S + S.T))
        cut = th[0] - (0.01 if r == 0 else 0.002)
        a0 = th[-1] + 0.01
        e = (cut + 1.0) / 2.0
        c0 = (cut - 1.0) / 2.0
        sigma1 = e / (a0 - c0)
        s_over_e = sigma1 / e
        w = v1 @ X
        T = (-c0 * s_over_e) * X - (s_over_e * v1)[:, None] * w[None, :]
        Y = _pmma(An, s_over_e * X, T)
        sig = sigma1
        for _ in range(2, DEG + 1):
            sig_new = 1.0 / (2.0 / sigma1 - sig)
            coef = 2.0 * sig_new / e
            wy = v1 @ Y
            T = (-c0 * coef) * Y - (coef * v1)[:, None] * wy[None, :] \
                - (sig * sig_new) * X
            Ynew = _pmma(An, coef * Y, T)
            X, Y, sig = Y, Ynew, sig_new
        X = _cholqr2(Y)

    AX = _apply_deflated(An, v1, X)
    S = _pmm(X.T, AX, bm=M_SUB, bn=M_SUB)
    _, Uvec = jnp.linalg.eigh(0.5 * (S + S.T))
    B = jnp.pad(Uvec[:, M_SUB - (FREQ - 1):], ((0, 0), (1, 0)))
    col0 = (jnp.arange(FREQ) == 0).astype(jnp.float32)
    vec = _pmma(X, B, v1[:, None] * col0[None, :], bm=512)

    R_b = jnp.concatenate([Cu0, R], axis=1)
    d_rb = R_b.sum(axis=0)
    d_rb_inv = d_rb ** -0.5
    d_rb_pos = d_rb ** 0.5
    P0 = _pmm(R * di_inv[None, :], Ci0) * di_pos[None, :]
    Z1 = _pmm(R_b * d_rb_inv[None, :], vec, bn=FREQ)
    P11 = vec[user_num:, :].T * d_rb_pos[None, user_num:]
    P = _pmma(Z1, (1.0 - PHI) * P11, PHI * P0, bm=512)
    return P


# speedup vs baseline: 46.0011x; 3.4303x over previous
import functools

import jax
import jax.numpy as jnp
from jax.experimental import pallas as pl

FREQ = 256
PHI = 0.7
M_SUB = 384
ROUNDS = 3
DEG = 20
CUTS = (0.17, 0.19, 0.19)
A0 = 0.35


def _mm_kernel(a_ref, b_ref, o_ref):
    o_ref[...] = jax.lax.dot_general(
        a_ref[...], b_ref[...], (((1,), (0,)), ((), ())),
        preferred_element_type=jnp.float32,
        precision=jax.lax.Precision.HIGHEST,
    )


def _mma_kernel(a_ref, b_ref, t_ref, o_ref):
    o_ref[...] = t_ref[...] + jax.lax.dot_general(
        a_ref[...], b_ref[...], (((1,), (0,)), ((), ())),
        preferred_element_type=jnp.float32,
        precision=jax.lax.Precision.HIGHEST,
    )


def _mma_kernel_fast(a_ref, b_ref, t_ref, o_ref):
    o_ref[...] = t_ref[...] + jax.lax.dot_general(
        a_ref[...], b_ref[...], (((1,), (0,)), ((), ())),
        preferred_element_type=jnp.float32,
        precision=jax.lax.Precision.DEFAULT,
    )


@functools.partial(jax.jit, static_argnames=("bm", "bn"))
def _pmm(a, b, bm=512, bn=512):
    M, K = a.shape
    _, N = b.shape
    bn = min(bn, N)
    return pl.pallas_call(
        _mm_kernel,
        grid=(M // bm, N // bn),
        in_specs=[
            pl.BlockSpec((bm, K), lambda i, j: (i, 0)),
            pl.BlockSpec((K, bn), lambda i, j: (0, j)),
        ],
        out_specs=pl.BlockSpec((bm, bn), lambda i, j: (i, j)),
        out_shape=jax.ShapeDtypeStruct((M, N), jnp.float32),
    )(a, b)


@functools.partial(jax.jit, static_argnames=("bm", "fast"))
def _pmma(a, b, t, bm=512, fast=False):
    M, K = a.shape
    _, N = b.shape
    return pl.pallas_call(
        _mma_kernel_fast if fast else _mma_kernel,
        grid=(M // bm,),
        in_specs=[
            pl.BlockSpec((bm, K), lambda i: (i, 0)),
            pl.BlockSpec((K, N), lambda i: (0, 0)),
            pl.BlockSpec((bm, N), lambda i: (i, 0)),
        ],
        out_specs=pl.BlockSpec((bm, N), lambda i: (i, 0)),
        out_shape=jax.ShapeDtypeStruct((M, N), jnp.float32),
    )(a, b, t)


def _cholqr(X, passes):
    for _ in range(passes):
        G = _pmm(X.T, X, bm=M_SUB, bn=M_SUB)
        L = jnp.linalg.cholesky(G)
        Linv = jax.lax.linalg.triangular_solve(
            L, jnp.eye(M_SUB, dtype=jnp.float32), lower=True, left_side=True)
        X = _pmm(X, Linv.T, bm=512, bn=M_SUB)
    return X


def kernel(R):
    user_num, item_num = R.shape
    n_tot = user_num + item_num
    du = R.sum(axis=1)
    di = R.sum(axis=0)
    du_inv = du ** -0.5
    di_inv = di ** -0.5
    di_pos = di ** 0.5
    Ru = du_inv[:, None] * R
    Ri = R * di_inv[None, :]
    R_post = Ru * di_inv[None, :]

    Cu = _pmm(Ri, Ri.T)
    Ci = _pmm(Ru.T, Ru)
    Ci0 = _pmm(R_post.T, R_post)
    Cu0 = _pmm(R_post, R_post.T)
    A = jnp.block([[Cu, R], [R.T, Ci]])
    dA = A.sum(axis=0)
    dA_inv = dA ** -0.5
    An = dA_inv[:, None] * A * dA_inv[None, :]
    Anb = An.astype(jnp.bfloat16)
    v1 = jnp.sqrt(dA)
    v1 = v1 / jnp.linalg.norm(v1)

    X = jax.random.normal(jax.random.key(0), (n_tot, M_SUB), dtype=jnp.float32)
    for r in range(ROUNDS):
        cut = CUTS[min(r, len(CUTS) - 1)]
        e = (cut + 1.0) / 2.0
        c0 = (cut - 1.0) / 2.0
        sigma1 = e / (A0 - c0)
        s_over_e = sigma1 / e
        w = v1 @ X
        T = (-c0 * s_over_e) * X - (s_over_e * v1)[:, None] * w[None, :]
        Y = _pmma(Anb, (s_over_e * X).astype(jnp.bfloat16), T, fast=True)
        sig = sigma1
        for _ in range(2, DEG + 1):
            sig_new = 1.0 / (2.0 / sigma1 - sig)
            coef = 2.0 * sig_new / e
            wy = v1 @ Y
            T = (-c0 * coef) * Y - (coef * v1)[:, None] * wy[None, :] \
                - (sig * sig_new) * X
            Ynew = _pmma(Anb, (coef * Y).astype(jnp.bfloat16), T, fast=True)
            X, Y, sig = Y, Ynew, sig_new
        X = _cholqr(Y, passes=(2 if r == ROUNDS - 1 else 1))

    wx = v1 @ X
    AX = _pmma(An, X, -v1[:, None] * wx[None, :])
    S = _pmm(X.T, AX, bm=M_SUB, bn=M_SUB)
    _, Uvec = jnp.linalg.eigh(0.5 * (S + S.T))
    B = jnp.pad(Uvec[:, M_SUB - (FREQ - 1):], ((0, 0), (1, 0)))
    col0 = (jnp.arange(FREQ) == 0).astype(jnp.float32)
    vec = _pmma(X, B, v1[:, None] * col0[None, :], bm=512)

    R_b = jnp.concatenate([Cu0, R], axis=1)
    d_rb = R_b.sum(axis=0)
    d_rb_inv = d_rb ** -0.5
    d_rb_pos = d_rb ** 0.5
    P0 = _pmm(R * di_inv[None, :], Ci0) * di_pos[None, :]
    Z1 = _pmm(R_b * d_rb_inv[None, :], vec, bn=FREQ)
    P11 = vec[user_num:, :].T * d_rb_pos[None, user_num:]
    P = _pmma(Z1, (1.0 - PHI) * P11, PHI * P0, bm=512)
    return P


# speedup vs baseline: 48.0057x; 1.0436x over previous
import functools

import jax
import jax.numpy as jnp
from jax.experimental import pallas as pl

FREQ = 256
PHI = 0.7
M_SUB = 384
ROUNDS = 3
DEG = 20
CUTS = (0.17, 0.19, 0.19)
A0 = 0.35


def _mm_kernel(a_ref, b_ref, o_ref):
    o_ref[...] = jax.lax.dot_general(
        a_ref[...], b_ref[...], (((1,), (0,)), ((), ())),
        preferred_element_type=jnp.float32,
        precision=jax.lax.Precision.HIGHEST,
    )


def _mma_kernel(a_ref, b_ref, t_ref, o_ref):
    o_ref[...] = t_ref[...] + jax.lax.dot_general(
        a_ref[...], b_ref[...], (((1,), (0,)), ((), ())),
        preferred_element_type=jnp.float32,
        precision=jax.lax.Precision.HIGHEST,
    )


@functools.partial(jax.jit, static_argnames=("bm", "bn"))
def _pmm(a, b, bm=512, bn=512):
    M, K = a.shape
    _, N = b.shape
    bn = min(bn, N)
    return pl.pallas_call(
        _mm_kernel,
        grid=(M // bm, N // bn),
        in_specs=[
            pl.BlockSpec((bm, K), lambda i, j: (i, 0)),
            pl.BlockSpec((K, bn), lambda i, j: (0, j)),
        ],
        out_specs=pl.BlockSpec((bm, bn), lambda i, j: (i, j)),
        out_shape=jax.ShapeDtypeStruct((M, N), jnp.float32),
    )(a, b)


@functools.partial(jax.jit, static_argnames=("bm",))
def _pmma(a, b, t, bm=512):
    M, K = a.shape
    _, N = b.shape
    return pl.pallas_call(
        _mma_kernel,
        grid=(M // bm,),
        in_specs=[
            pl.BlockSpec((bm, K), lambda i: (i, 0)),
            pl.BlockSpec((K, N), lambda i: (0, 0)),
            pl.BlockSpec((bm, N), lambda i: (i, 0)),
        ],
        out_specs=pl.BlockSpec((bm, N), lambda i: (i, 0)),
        out_shape=jax.ShapeDtypeStruct((M, N), jnp.float32),
    )(a, b, t)


def _cheb_step(Anb, Y, X, v1c, w8, cy, cx, cm, bm=512):
    M, K = Anb.shape
    _, N = Y.shape

    def body(anb_ref, yfull_ref, y_ref, x_ref, v1_ref, w_ref, o_ref):
        yb = yfull_ref[...].astype(jnp.bfloat16)
        acc = jax.lax.dot_general(
            anb_ref[...], yb, (((1,), (0,)), ((), ())),
            preferred_element_type=jnp.float32,
            precision=jax.lax.Precision.DEFAULT,
        )
        o_ref[...] = (cm * acc + cy * y_ref[...] + cx * x_ref[...]
                      - cm * v1_ref[...] * w_ref[0:1, :])

    return pl.pallas_call(
        body,
        grid=(M // bm,),
        in_specs=[
            pl.BlockSpec((bm, K), lambda i: (i, 0)),
            pl.BlockSpec((K, N), lambda i: (0, 0)),
            pl.BlockSpec((bm, N), lambda i: (i, 0)),
            pl.BlockSpec((bm, N), lambda i: (i, 0)),
            pl.BlockSpec((bm, 1), lambda i: (i, 0)),
            pl.BlockSpec((8, N), lambda i: (0, 0)),
        ],
        out_specs=pl.BlockSpec((bm, N), lambda i: (i, 0)),
        out_shape=jax.ShapeDtypeStruct((M, N), jnp.float32),
    )(Anb, Y, Y, X, v1c, w8)


def _cholqr(X, passes=1):
    for _ in range(passes):
        G = _pmm(X.T, X, bm=M_SUB, bn=M_SUB)
        G = G + (1e-6 * jnp.trace(G) / M_SUB) * jnp.eye(M_SUB, dtype=jnp.float32)
        L = jnp.linalg.cholesky(G)
        Linv = jax.lax.linalg.triangular_solve(
            L, jnp.eye(M_SUB, dtype=jnp.float32), lower=True, left_side=True)
        X = _pmm(X, Linv.T, bm=512, bn=M_SUB)
    return X


def kernel(R):
    user_num, item_num = R.shape
    n_tot = user_num + item_num
    du = R.sum(axis=1)
    di = R.sum(axis=0)
    du_inv = du ** -0.5
    di_inv = di ** -0.5
    di_pos = di ** 0.5
    Ru = du_inv[:, None] * R
    Ri = R * di_inv[None, :]
    R_post = Ru * di_inv[None, :]

    Cu = _pmm(Ri, Ri.T)
    Ci = _pmm(Ru.T, Ru)
    Ci0 = _pmm(R_post.T, R_post)
    Cu0 = _pmm(R_post, R_post.T)
    A = jnp.block([[Cu, R], [R.T, Ci]])
    dA = A.sum(axis=0)
    dA_inv = dA ** -0.5
    An = dA_inv[:, None] * A * dA_inv[None, :]
    Anb = An.astype(jnp.bfloat16)
    v1 = jnp.sqrt(dA)
    v1 = v1 / jnp.linalg.norm(v1)
    v1c = v1[:, None]

    X = jax.random.normal(jax.random.key(0), (n_tot, M_SUB), dtype=jnp.float32)
    for r in range(ROUNDS):
        cut = CUTS[min(r, len(CUTS) - 1)]
        e = (cut + 1.0) / 2.0
        c0 = (cut - 1.0) / 2.0
        sigma1 = e / (A0 - c0)
        s_over_e = sigma1 / e
        w8 = jnp.broadcast_to((v1 @ X)[None, :], (8, M_SUB))
        Y = _cheb_step(Anb, X, X, v1c, w8,
                       cy=0.0, cx=(-c0 * s_over_e), cm=s_over_e)
        sig = sigma1
        for _ in range(2, DEG + 1):
            sig_new = 1.0 / (2.0 / sigma1 - sig)
            coef = 2.0 * sig_new / e
            w8 = jnp.broadcast_to((v1 @ Y)[None, :], (8, M_SUB))
            Ynew = _cheb_step(Anb, Y, X, v1c, w8,
                              cy=(-c0 * coef), cx=(-sig * sig_new), cm=coef)
            X, Y, sig = Y, Ynew, sig_new
        X = _cholqr(Y, passes=1)

    wx = v1 @ X
    AX = _pmma(An, X, -v1c * wx[None, :])
    S = _pmm(X.T, AX, bm=M_SUB, bn=M_SUB)
    _, Uvec = jnp.linalg.eigh(0.5 * (S + S.T))
    B = jnp.pad(Uvec[:, M_SUB - (FREQ - 1):], ((0, 0), (1, 0)))
    col0 = (jnp.arange(FREQ) == 0).astype(jnp.float32)
    vec = _pmma(X, B, v1c * col0[None, :], bm=512)

    R_b = jnp.concatenate([Cu0, R], axis=1)
    d_rb = R_b.sum(axis=0)
    d_rb_inv = d_rb ** -0.5
    d_rb_pos = d_rb ** 0.5
    P0 = _pmm(R * di_inv[None, :], Ci0) * di_pos[None, :]
    Z1 = _pmm(R_b * d_rb_inv[None, :], vec, bn=FREQ)
    P11 = vec[user_num:, :].T * d_rb_pos[None, user_num:]
    P = _pmma(Z1, (1.0 - PHI) * P11, PHI * P0, bm=512)
    return P


# speedup vs baseline: 53.8893x; 1.1226x over previous
import functools

import jax
import jax.numpy as jnp
from jax.experimental import pallas as pl

FREQ = 256
PHI = 0.7
M_SUB = 384
ROUNDS = 3
DEG = 16
CUTS = (0.17, 0.19, 0.19)
A0 = 0.35


def _mm_kernel(a_ref, b_ref, o_ref):
    o_ref[...] = jax.lax.dot_general(
        a_ref[...], b_ref[...], (((1,), (0,)), ((), ())),
        preferred_element_type=jnp.float32,
        precision=jax.lax.Precision.HIGHEST,
    )


def _mm_kernel_fast(a_ref, b_ref, o_ref):
    o_ref[...] = jax.lax.dot_general(
        a_ref[...], b_ref[...], (((1,), (0,)), ((), ())),
        preferred_element_type=jnp.float32,
        precision=jax.lax.Precision.DEFAULT,
    )


def _mma_kernel(a_ref, b_ref, t_ref, o_ref):
    o_ref[...] = t_ref[...] + jax.lax.dot_general(
        a_ref[...], b_ref[...], (((1,), (0,)), ((), ())),
        preferred_element_type=jnp.float32,
        precision=jax.lax.Precision.HIGHEST,
    )


def _mma_kernel_fast(a_ref, b_ref, t_ref, o_ref):
    o_ref[...] = t_ref[...] + jax.lax.dot_general(
        a_ref[...], b_ref[...], (((1,), (0,)), ((), ())),
        preferred_element_type=jnp.float32,
        precision=jax.lax.Precision.DEFAULT,
    )


@functools.partial(jax.jit, static_argnames=("bm", "bn", "fast"))
def _pmm(a, b, bm=512, bn=512, fast=False):
    M, K = a.shape
    _, N = b.shape
    bn = min(bn, N)
    return pl.pallas_call(
        _mm_kernel_fast if fast else _mm_kernel,
        grid=(M // bm, N // bn),
        in_specs=[
            pl.BlockSpec((bm, K), lambda i, j: (i, 0)),
            pl.BlockSpec((K, bn), lambda i, j: (0, j)),
        ],
        out_specs=pl.BlockSpec((bm, bn), lambda i, j: (i, j)),
        out_shape=jax.ShapeDtypeStruct((M, N), jnp.float32),
    )(a, b)


@functools.partial(jax.jit, static_argnames=("bm", "fast"))
def _pmma(a, b, t, bm=512, fast=False):
    M, K = a.shape
    _, N = b.shape
    return pl.pallas_call(
        _mma_kernel_fast if fast else _mma_kernel,
        grid=(M // bm,),
        in_specs=[
            pl.BlockSpec((bm, K), lambda i: (i, 0)),
            pl.BlockSpec((K, N), lambda i: (0, 0)),
            pl.BlockSpec((bm, N), lambda i: (i, 0)),
        ],
        out_specs=pl.BlockSpec((bm, N), lambda i: (i, 0)),
        out_shape=jax.ShapeDtypeStruct((M, N), jnp.float32),
    )(a, b, t)


def _cheb_step(Anb, Y, X, v1c, w8, cy, cx, cm, bm=512):
    M, K = Anb.shape
    _, N = Y.shape

    def body(anb_ref, yfull_ref, y_ref, x_ref, v1_ref, w_ref, o_ref):
        yb = yfull_ref[...].astype(jnp.bfloat16)
        acc = jax.lax.dot_general(
            anb_ref[...], yb, (((1,), (0,)), ((), ())),
            preferred_element_type=jnp.float32,
            precision=jax.lax.Precision.DEFAULT,
        )
        o_ref[...] = (cm * acc + cy * y_ref[...] + cx * x_ref[...]
                      - cm * v1_ref[...] * w_ref[0:1, :])

    return pl.pallas_call(
        body,
        grid=(M // bm,),
        in_specs=[
            pl.BlockSpec((bm, K), lambda i: (i, 0)),
            pl.BlockSpec((K, N), lambda i: (0, 0)),
            pl.BlockSpec((bm, N), lambda i: (i, 0)),
            pl.BlockSpec((bm, N), lambda i: (i, 0)),
            pl.BlockSpec((bm, 1), lambda i: (i, 0)),
            pl.BlockSpec((8, N), lambda i: (0, 0)),
        ],
        out_specs=pl.BlockSpec((bm, N), lambda i: (i, 0)),
        out_shape=jax.ShapeDtypeStruct((M, N), jnp.float32),
    )(Anb, Y, Y, X, v1c, w8)


def _cholqr(X, passes=1):
    for _ in range(passes):
        G = _pmm(X.T, X, bm=M_SUB, bn=M_SUB)
        G = G + (1e-6 * jnp.trace(G) / M_SUB) * jnp.eye(M_SUB, dtype=jnp.float32)
        L = jnp.linalg.cholesky(G)
        Linv = jax.lax.linalg.triangular_solve(
            L, jnp.eye(M_SUB, dtype=jnp.float32), lower=True, left_side=True)
        X = _pmm(X, Linv.T, bm=512, bn=M_SUB)
    return X


def kernel(R):
    user_num, item_num = R.shape
    n_tot = user_num + item_num
    du = R.sum(axis=1)
    di = R.sum(axis=0)
    du_inv = du ** -0.5
    di_inv = di ** -0.5
    di_pos = di ** 0.5
    Ru = du_inv[:, None] * R
    Ri = R * di_inv[None, :]
    R_post = Ru * di_inv[None, :]

    Cu = _pmm(Ri, Ri.T)
    Ci = _pmm(Ru.T, Ru)
    Ci0 = _pmm(R_post.T, R_post, fast=True)
    Cu0 = _pmm(R_post, R_post.T, fast=True)
    A = jnp.block([[Cu, R], [R.T, Ci]])
    dA = A.sum(axis=0)
    dA_inv = dA ** -0.5
    An = dA_inv[:, None] * A * dA_inv[None, :]
    Anb = An.astype(jnp.bfloat16)
    v1 = jnp.sqrt(dA)
    v1 = v1 / jnp.linalg.norm(v1)
    v1c = v1[:, None]

    X = jax.random.normal(jax.random.key(0), (n_tot, M_SUB), dtype=jnp.float32)
    for r in range(ROUNDS):
        cut = CUTS[min(r, len(CUTS) - 1)]
        e = (cut + 1.0) / 2.0
        c0 = (cut - 1.0) / 2.0
        sigma1 = e / (A0 - c0)
        s_over_e = sigma1 / e
        w8 = jnp.broadcast_to((v1 @ X)[None, :], (8, M_SUB))
        Y = _cheb_step(Anb, X, X, v1c, w8,
                       cy=0.0, cx=(-c0 * s_over_e), cm=s_over_e)
        sig = sigma1
        for _ in range(2, DEG + 1):
            sig_new = 1.0 / (2.0 / sigma1 - sig)
            coef = 2.0 * sig_new / e
            w8 = jnp.broadcast_to((v1 @ Y)[None, :], (8, M_SUB))
            Ynew = _cheb_step(Anb, Y, X, v1c, w8,
                              cy=(-c0 * coef), cx=(-sig * sig_new), cm=coef)
            X, Y, sig = Y, Ynew, sig_new
        X = _cholqr(Y, passes=1)

    wx = v1 @ X
    AX = _pmma(An, X, -v1c * wx[None, :])
    S = _pmm(X.T, AX, bm=M_SUB, bn=M_SUB)
    _, Uvec = jnp.linalg.eigh(0.5 * (S + S.T))
    B = jnp.pad(Uvec[:, M_SUB - (FREQ - 1):], ((0, 0), (1, 0)))
    col0 = (jnp.arange(FREQ) == 0).astype(jnp.float32)
    vec = _pmma(X, B, v1c * col0[None, :], bm=512, fast=True)

    R_b = jnp.concatenate([Cu0, R], axis=1)
    d_rb = R_b.sum(axis=0)
    d_rb_inv = d_rb ** -0.5
    d_rb_pos = d_rb ** 0.5
    P0 = _pmm(R * di_inv[None, :], Ci0, fast=True) * di_pos[None, :]
    Z1 = _pmm(R_b * d_rb_inv[None, :], vec, bn=FREQ, fast=True)
    P11 = vec[user_num:, :].T * d_rb_pos[None, user_num:]
    P = _pmma(Z1, (1.0 - PHI) * P11, PHI * P0, bm=512, fast=True)
    return P
